# Initial kernel scaffold; baseline (speedup 1.0000x reference)
import jax
import jax.numpy as jnp
from jax.experimental import pallas as pl


def kernel(x, node_attr, edge_src, edge_dst, edge_attr, edge_length_embedding, batch, params):
    raise NotImplementedError("write your pallas kernel here")


# speedup vs baseline: 1.9122x; 1.9122x over previous
import numpy as np
import jax
import jax.numpy as jnp
from jax import lax
from jax.experimental import pallas as pl
from jax.experimental.pallas import tpu as pltpu
from jax.experimental.pallas import tpu_sc as plsc

_N = 10000
_NG = 64
_NPAD = 10240
_E = 320000
_EPAD = 323584
_C = 128
_CF = 64
_H = 144
_INV = 1.0 / np.sqrt(32.0)

_MESH = plsc.VectorSubcoreMesh(
    core_axis_name="c", subcore_axis_name="s", num_cores=2, num_subcores=16)
_SC_PARAMS = pltpu.CompilerParams(use_tc_tiling_on_sc=False)

_f32 = jnp.float32


def _silu(v):
    return v * lax.logistic(v)


def _radial_call(emb, W0, W1, W2, W3, split):
    BE = 1024
    d1 = W3.shape[1]

    def body(emb_ref, w0_ref, w1_ref, w2_ref, w3_ref, *outs):
        v = _silu(jnp.dot(emb_ref[...], w0_ref[...], preferred_element_type=_f32))
        v = _silu(jnp.dot(v, w1_ref[...], preferred_element_type=_f32))
        v = _silu(jnp.dot(v, w2_ref[...], preferred_element_type=_f32))
        v = jnp.dot(v, w3_ref[...], preferred_element_type=_f32)
        if split:
            outs[0][...] = v[:, :_H]
            outs[1][...] = v[:, _H:]
        else:
            outs[0][...] = v

    def full(shp):
        return pl.BlockSpec(shp, lambda i: (0, 0))

    in_specs = [pl.BlockSpec((BE, emb.shape[1]), lambda i: (i, 0)),
                full(W0.shape), full(W1.shape), full(W2.shape), full(W3.shape)]
    if split:
        out_shape = [jax.ShapeDtypeStruct((_EPAD, _H), _f32)] * 2
        out_specs = [pl.BlockSpec((BE, _H), lambda i: (i, 0))] * 2
    else:
        out_shape = [jax.ShapeDtypeStruct((_EPAD, d1), _f32)]
        out_specs = [pl.BlockSpec((BE, d1), lambda i: (i, 0))]
    return pl.pallas_call(
        body, grid=(_EPAD // BE,), in_specs=in_specs, out_specs=out_specs,
        out_shape=out_shape)(emb, W0, W1, W2, W3)


def _msg_split_call(h, W):
    BN = 256

    def body(h_ref, w_ref, o0, o1):
        hm = jnp.dot(h_ref[...], w_ref[...], preferred_element_type=_f32)
        o0[...] = hm[:, :_H]
        o1[...] = hm[:, _H:]

    return pl.pallas_call(
        body, grid=(_NPAD // BN,),
        in_specs=[pl.BlockSpec((BN, h.shape[1]), lambda i: (i, 0)),
                  pl.BlockSpec(W.shape, lambda i: (0, 0))],
        out_specs=[pl.BlockSpec((BN, _H), lambda i: (i, 0))] * 2,
        out_shape=[jax.ShapeDtypeStruct((_NPAD, _H), _f32)] * 2)(h, W)


def _node_call(h, agg_a, agg_b, ea0, ea1, na, Wself, WedgeP, Wattr, Wmsg_next,
               *, cat, do_silu, split_next):
    BN = 256
    d1 = Wself.shape[1]
    ha = agg_a.shape[1]

    def body(h_ref, aa_ref, ab_ref, e0_ref, e1_ref, na_ref, ws_ref, we_ref,
             wa_ref, *rest):
        if Wmsg_next is not None:
            wm_ref, outs = rest[0], rest[1:]
        else:
            outs = rest
        if cat:
            agg = jnp.concatenate([aa_ref[...], ab_ref[...]], axis=1)
        else:
            agg = aa_ref[...] + ab_ref[...]
        ea = e0_ref[...] + e1_ref[...]
        z = jnp.dot(h_ref[...], ws_ref[...], preferred_element_type=_f32)
        z = z + (agg + jnp.dot(ea, we_ref[...], preferred_element_type=_f32)) * _INV
        z = z + na_ref[...][:, :1] * wa_ref[...]
        if do_silu:
            z = _silu(z)
        outs[0][...] = z
        if Wmsg_next is not None:
            hm = jnp.dot(z, wm_ref[...], preferred_element_type=_f32)
            if split_next:
                outs[1][...] = hm[:, :_H]
                outs[2][...] = hm[:, _H:]
            else:
                outs[1][...] = hm

    def full(shp):
        return pl.BlockSpec(shp, lambda i: (0, 0))

    in_specs = [pl.BlockSpec((BN, h.shape[1]), lambda i: (i, 0)),
                pl.BlockSpec((BN, ha), lambda i: (i, 0)),
                pl.BlockSpec((BN, ha), lambda i: (i, 0)),
                pl.BlockSpec((BN, 16), lambda i: (i, 0)),
                pl.BlockSpec((BN, 16), lambda i: (i, 0)),
                pl.BlockSpec((BN, 16), lambda i: (i, 0)),
                full(Wself.shape), full(WedgeP.shape), full(Wattr.shape)]
    args = [h, agg_a, agg_b, ea0, ea1, na, Wself, WedgeP, Wattr]
    out_shape = [jax.ShapeDtypeStruct((_NPAD, d1), _f32)]
    out_specs = [pl.BlockSpec((BN, d1), lambda i: (i, 0))]
    if Wmsg_next is not None:
        in_specs.append(full(Wmsg_next.shape))
        args.append(Wmsg_next)
        if split_next:
            out_shape += [jax.ShapeDtypeStruct((_NPAD, _H), _f32)] * 2
            out_specs += [pl.BlockSpec((BN, _H), lambda i: (i, 0))] * 2
        else:
            dn = Wmsg_next.shape[1]
            out_shape.append(jax.ShapeDtypeStruct((_NPAD, dn), _f32))
            out_specs.append(pl.BlockSpec((BN, dn), lambda i: (i, 0)))
    return pl.pallas_call(
        body, grid=(_NPAD // BN,), in_specs=in_specs, out_specs=out_specs,
        out_shape=out_shape)(*args)


def _softmax_call(pool):
    def body(p_ref, o_ref):
        p = p_ref[0] + p_ref[1]
        s = p[:_NG, :10]
        m = jnp.max(s, axis=1, keepdims=True)
        e = jnp.exp(s - m)
        o_ref[...] = e / jnp.sum(e, axis=1, keepdims=True)

    return pl.pallas_call(
        body, out_shape=jax.ShapeDtypeStruct((_NG, 10), _f32))(pool)


def _edge_fs_call(hm0, hm1, w0, w1, srcp, dstp):
    NCH = _EPAD // (16 * _CF)
    RPT = _NPAD // 16

    def body(hm0_ref, hm1_ref, w0_ref, w1_ref, src_ref, dst_ref,
             out0_ref, out1_ref, sidx, didx, rows, wbuf, acc, sem):
        c = lax.axis_index("c")
        s = lax.axis_index("s")
        z16 = jnp.zeros((16,), _f32)

        def zrow(i, cc):
            for j in range(_H // 16):
                wbuf[i, pl.ds(j * 16, 16)] = z16
            return cc
        lax.fori_loop(0, _CF, zrow, 0)
        for t in range(RPT // _CF):
            pltpu.sync_copy(wbuf, acc.at[pl.ds(s * RPT + t * _CF, _CF)])
        plsc.subcore_barrier()

        def chunk(k, cc):
            base = (s * NCH + k) * _CF
            pltpu.sync_copy(src_ref.at[pl.ds(base, _CF)], sidx)
            pltpu.sync_copy(dst_ref.at[pl.ds(base, _CF)], didx)

            @pl.when(c == 0)
            def _():
                cp = pltpu.async_copy(hm0_ref.at[sidx], rows, sem)
                pltpu.sync_copy(w0_ref.at[pl.ds(base, _CF)], wbuf)
                cp.wait()

            @pl.when(c == 1)
            def _():
                cp = pltpu.async_copy(hm1_ref.at[sidx], rows, sem)
                pltpu.sync_copy(w1_ref.at[pl.ds(base, _CF)], wbuf)
                cp.wait()

            def mrow(i, c2):
                for j in range(_H // 16):
                    sl = pl.ds(j * 16, 16)
                    rows[i, sl] = rows[i, sl] * wbuf[i, sl]
                return c2
            lax.fori_loop(0, _CF, mrow, 0)
            pltpu.sync_copy(rows, acc.at[didx], add=True)
            return cc
        lax.fori_loop(0, NCH, chunk, 0)
        plsc.subcore_barrier()

        for t in range(RPT // _CF):
            sl = pl.ds(s * RPT + t * _CF, _CF)

            @pl.when(c == 0)
            def _():
                pltpu.sync_copy(acc.at[sl], out0_ref.at[sl])

            @pl.when(c == 1)
            def _():
                pltpu.sync_copy(acc.at[sl], out1_ref.at[sl])

    return pl.kernel(
        body,
        out_type=[jax.ShapeDtypeStruct((_NPAD, _H), _f32)] * 2,
        mesh=_MESH,
        compiler_params=_SC_PARAMS,
        scratch_types=[
            pltpu.VMEM((_CF,), jnp.int32),
            pltpu.VMEM((_CF,), jnp.int32),
            pltpu.VMEM((_CF, _H), _f32),
            pltpu.VMEM((_CF, _H), _f32),
            pltpu.VMEM_SHARED((_NPAD, _H), _f32),
            pltpu.SemaphoreType.DMA,
        ])(hm0, hm1, w0, w1, srcp, dstp)


def _edge_es_call(hm4, w4, srcp, dstp):
    NCH = _EPAD // (32 * _C)
    RPT = _NPAD // 16

    def body(hm_ref, w_ref, src_ref, dst_ref, out_ref,
             sidx, didx, rows, wbuf, acc, sem):
        c = lax.axis_index("c")
        s = lax.axis_index("s")
        wid = c * 16 + s
        z16 = jnp.zeros((16,), _f32)

        def zrow(i, cc):
            rows[i, pl.ds(0, 16)] = z16
            return cc
        lax.fori_loop(0, _C, zrow, 0)
        for t in range(RPT // _C):
            pltpu.sync_copy(rows, acc.at[pl.ds(s * RPT + t * _C, _C)])
        plsc.subcore_barrier()

        def chunk(k, cc):
            base = (wid * NCH + k) * _C
            pltpu.sync_copy(src_ref.at[pl.ds(base, _C)], sidx)
            pltpu.sync_copy(dst_ref.at[pl.ds(base, _C)], didx)
            cp = pltpu.async_copy(hm_ref.at[sidx], rows, sem)
            pltpu.sync_copy(w_ref.at[pl.ds(base, _C)], wbuf)
            cp.wait()

            def mrow(i, c2):
                sl = pl.ds(0, 16)
                rows[i, sl] = rows[i, sl] * wbuf[i, sl]
                return c2
            lax.fori_loop(0, _C, mrow, 0)
            pltpu.sync_copy(rows, acc.at[didx], add=True)
            return cc
        lax.fori_loop(0, NCH, chunk, 0)
        plsc.subcore_barrier()

        for t in range(RPT // _C):
            sl = pl.ds(s * RPT + t * _C, _C)
            pltpu.sync_copy(acc.at[sl], out_ref.at[c, sl])

    return pl.kernel(
        body,
        out_type=jax.ShapeDtypeStruct((2, _NPAD, 16), _f32),
        mesh=_MESH,
        compiler_params=_SC_PARAMS,
        scratch_types=[
            pltpu.VMEM((_C,), jnp.int32),
            pltpu.VMEM((_C,), jnp.int32),
            pltpu.VMEM((_C, 16), _f32),
            pltpu.VMEM((_C, 16), _f32),
            pltpu.VMEM_SHARED((_NPAD, 16), _f32),
            pltpu.SemaphoreType.DMA,
        ])(hm4, w4, srcp, dstp)


def _ea_call(eap, dstp):
    NCH = _EPAD // (32 * _C)
    RPT = _NPAD // 16

    def body(ea_ref, dst_ref, out_ref, didx, rows, acc, sem):
        c = lax.axis_index("c")
        s = lax.axis_index("s")
        wid = c * 16 + s
        z16 = jnp.zeros((16,), _f32)

        def zrow(i, cc):
            rows[i, pl.ds(0, 16)] = z16
            return cc
        lax.fori_loop(0, _C, zrow, 0)
        for t in range(RPT // _C):
            pltpu.sync_copy(rows, acc.at[pl.ds(s * RPT + t * _C, _C)])
        plsc.subcore_barrier()

        def chunk(k, cc):
            base = (wid * NCH + k) * _C
            pltpu.sync_copy(dst_ref.at[pl.ds(base, _C)], didx)
            pltpu.sync_copy(ea_ref.at[pl.ds(base, _C)], rows)
            pltpu.sync_copy(rows, acc.at[didx], add=True)
            return cc
        lax.fori_loop(0, NCH, chunk, 0)
        plsc.subcore_barrier()

        for t in range(RPT // _C):
            sl = pl.ds(s * RPT + t * _C, _C)
            pltpu.sync_copy(acc.at[sl], out_ref.at[c, sl])

    return pl.kernel(
        body,
        out_type=jax.ShapeDtypeStruct((2, _NPAD, 16), _f32),
        mesh=_MESH,
        compiler_params=_SC_PARAMS,
        scratch_types=[
            pltpu.VMEM((_C,), jnp.int32),
            pltpu.VMEM((_C, 16), _f32),
            pltpu.VMEM_SHARED((_NPAD, 16), _f32),
            pltpu.SemaphoreType.DMA,
        ])(eap, dstp)


def _pool_call(h4, batchp):
    CP = 64
    NCH = _NPAD // (32 * CP)

    def body(h_ref, b_ref, out_ref, bidx, rows, zbuf, acc, sem):
        c = lax.axis_index("c")
        s = lax.axis_index("s")
        wid = c * 16 + s
        z16 = jnp.zeros((16,), _f32)

        @pl.when(s == 0)
        def _():
            def zrow(i, cc):
                zbuf[i, pl.ds(0, 16)] = z16
                return cc
            lax.fori_loop(0, 72, zrow, 0)
            pltpu.sync_copy(zbuf, acc)
        plsc.subcore_barrier()

        def chunk(k, cc):
            base = (wid * NCH + k) * CP
            pltpu.sync_copy(b_ref.at[pl.ds(base, CP)], bidx)
            pltpu.sync_copy(h_ref.at[pl.ds(base, CP)], rows)
            pltpu.sync_copy(rows, acc.at[bidx], add=True)
            return cc
        lax.fori_loop(0, NCH, chunk, 0)
        plsc.subcore_barrier()

        @pl.when(s == 0)
        def _():
            pltpu.sync_copy(acc, out_ref.at[c])

    return pl.kernel(
        body,
        out_type=jax.ShapeDtypeStruct((2, 72, 16), _f32),
        mesh=_MESH,
        compiler_params=_SC_PARAMS,
        scratch_types=[
            pltpu.VMEM((CP,), jnp.int32),
            pltpu.VMEM((CP, 16), _f32),
            pltpu.VMEM((72, 16), _f32),
            pltpu.VMEM_SHARED((72, 16), _f32),
            pltpu.SemaphoreType.DMA,
        ])(h4, batchp)


def kernel(x, node_attr, edge_src, edge_dst, edge_attr, edge_length_embedding,
           batch, params):
    xp = jnp.zeros((_NPAD, 128), _f32).at[:_N].set(x)
    nap = jnp.zeros((_NPAD, 16), _f32).at[:_N].set(
        jnp.broadcast_to(node_attr, (_N, 16)))
    srcp = jnp.full((_EPAD,), _N, jnp.int32).at[:_E].set(edge_src.astype(jnp.int32))
    dstp = jnp.full((_EPAD,), _N, jnp.int32).at[:_E].set(edge_dst.astype(jnp.int32))
    eap = jnp.zeros((_EPAD, 16), _f32).at[:_E, :9].set(edge_attr)
    embp = jnp.zeros((_EPAD, 10), _f32).at[:_E].set(edge_length_embedding)
    batchp = jnp.full((_NPAD,), _NG, jnp.int32).at[:_N].set(batch.astype(jnp.int32))

    def wedgeP(p):
        return jnp.zeros((16, p['W_edge'].shape[1]), _f32).at[:9].set(p['W_edge'])

    p4 = params[3]
    Wself4 = jnp.zeros((288, 16), _f32).at[:, :10].set(p4['W_self'])
    Wedge4 = jnp.zeros((16, 16), _f32).at[:9, :10].set(p4['W_edge'])
    Wattr4 = jnp.zeros((1, 16), _f32).at[:, :10].set(p4['W_attr'])
    fcW3_4 = jnp.zeros((128, 16), _f32).at[:, :10].set(p4['fc_W3'])
    Wmsg4 = jnp.zeros((288, 16), _f32).at[:, :10].set(p4['W_msg'])

    ea_pair = _ea_call(eap, dstp)
    ea0, ea1 = ea_pair[0], ea_pair[1]

    hm0, hm1 = _msg_split_call(xp, params[0]['W_msg'])
    h = xp
    for li in range(3):
        p = params[li]
        w0, w1 = _radial_call(embp, p['fc_W0'], p['fc_W1'], p['fc_W2'],
                              p['fc_W3'], split=True)
        agg0, agg1 = _edge_fs_call(hm0, hm1, w0, w1, srcp, dstp)
        if li < 2:
            h, hm0, hm1 = _node_call(
                h, agg0, agg1, ea0, ea1, nap, p['W_self'], wedgeP(p),
                p['W_attr'], params[li + 1]['W_msg'],
                cat=True, do_silu=True, split_next=True)
        else:
            h, hm4 = _node_call(
                h, agg0, agg1, ea0, ea1, nap, p['W_self'], wedgeP(p),
                p['W_attr'], Wmsg4, cat=True, do_silu=True, split_next=False)

    (w4,) = _radial_call(embp, p4['fc_W0'], p4['fc_W1'], p4['fc_W2'], fcW3_4,
                         split=False)
    agg4 = _edge_es_call(hm4, w4, srcp, dstp)
    (h4,) = _node_call(h, agg4[0], agg4[1], ea0, ea1, nap, Wself4, Wedge4,
                       Wattr4, None, cat=False, do_silu=False, split_next=False)
    pool = _pool_call(h4, batchp)
    return _softmax_call(pool)
